# baseline (speedup 1.0000x reference)
import jax
import jax.numpy as jnp
from jax import lax
from jax.experimental import pallas as pl
from jax.experimental.pallas import tpu as pltpu
from jax.experimental.pallas import tpu_sc as plsc

_B, _S, _D = 16384, 200, 64
_N2 = _B * _S // 2

_INFO = plsc.get_sparse_core_info()
_NC, _NS = _INFO.num_cores, _INFO.num_subcores
_NW = _NC * _NS
_PER_W = _N2 // _NW
_C = 400
_CHUNKS = _PER_W // _C
_NBUF = 2


def _sc_body(idx_hbm, tab_hbm, out_hbm,
             idx_v0, idx_v1, rows_v0, rows_v1, gsem0, gsem1, wsem0, wsem1):
    wid = lax.axis_index("s") * _NC + lax.axis_index("c")
    base = wid * _PER_W
    idx_v = (idx_v0, idx_v1)
    rows_v = (rows_v0, rows_v1)
    gsem = (gsem0, gsem1)
    wsem = (wsem0, wsem1)

    def fire(i, b):
        off = base + i * _C
        pltpu.sync_copy(idx_hbm.at[pl.ds(off, _C)], idx_v[b])
        pltpu.async_copy(tab_hbm.at[idx_v[b]], rows_v[b], gsem[b])

    for b in range(_NBUF):
        fire(b, b)

    def step(g, _):
        for b in range(_NBUF):
            i = g * _NBUF + b
            pltpu.make_async_copy(tab_hbm.at[idx_v[b]], rows_v[b],
                                  gsem[b]).wait()
            pltpu.async_copy(rows_v[b], out_hbm.at[pl.ds(base + i * _C, _C)],
                             wsem[b])

            @pl.when(i + _NBUF < _CHUNKS)
            def _():
                pltpu.make_async_copy(rows_v[b],
                                      out_hbm.at[pl.ds(base + i * _C, _C)],
                                      wsem[b]).wait()
                fire(i + _NBUF, b)
        return ()

    lax.fori_loop(0, _CHUNKS // _NBUF, step, ())
    for b in range(_NBUF):
        pltpu.make_async_copy(rows_v[b], out_hbm.at[pl.ds(base, _C)],
                              wsem[b]).wait()


def kernel(inputs, table):
    idx2 = inputs.astype(jnp.int32).reshape(_N2, 2)
    cidx = 7 * idx2[:, 0] + idx2[:, 1]
    tab49 = jnp.concatenate(
        [jnp.repeat(table, 7, axis=0), jnp.tile(table, (7, 1))], axis=1)
    out = pl.kernel(
        _sc_body,
        out_type=jax.ShapeDtypeStruct((_N2, 2 * _D), jnp.float32),
        mesh=plsc.VectorSubcoreMesh(core_axis_name="c", subcore_axis_name="s"),
        scratch_types=[
            pltpu.VMEM((_C,), jnp.int32),
            pltpu.VMEM((_C,), jnp.int32),
            pltpu.VMEM((_C, 2 * _D), jnp.float32),
            pltpu.VMEM((_C, 2 * _D), jnp.float32),
            pltpu.SemaphoreType.DMA,
            pltpu.SemaphoreType.DMA,
            pltpu.SemaphoreType.DMA,
            pltpu.SemaphoreType.DMA,
        ],
    )(cidx, tab49)
    return out.reshape(_B, _S, _D)
